# initial kernel scaffold (unmeasured)
import functools

import jax
import jax.numpy as jnp
from jax import lax
from jax.experimental import pallas as pl
from jax.experimental.pallas import tpu as pltpu

N_DEV = 4
B = 2
S = 1024
SP = S // N_DEV
D = 768
H = 4
DH = 64


def _rope(t):
    half = DH // 2
    pos = lax.broadcasted_iota(jnp.float32, (S, half), 0)
    j = lax.broadcasted_iota(jnp.float32, (S, half), 1)
    inv = jnp.exp(-jnp.log(10000.0) * (2.0 * j) / DH)
    ang = pos * inv
    cosb = jnp.cos(ang)
    sinb = jnp.sin(ang)
    cos = jnp.stack([cosb, cosb], axis=-1).reshape(S, DH)
    sin = jnp.stack([sinb, sinb], axis=-1).reshape(S, DH)
    cos2 = jnp.concatenate([cos] * B, axis=0)[:, None, :]
    sin2 = jnp.concatenate([sin] * B, axis=0)[:, None, :]
    t2 = t.reshape(B * S, H, half, 2)
    t_r = jnp.stack([-t2[..., 1], t2[..., 0]], axis=-1).reshape(B * S, H, DH)
    return t * cos2 + t_r * sin2


def kernel(x, Wq, Wk, Wv, Wo):
    def body(x_ref, wq_ref, wk_ref, wv_ref, wo_ref, out_ref,
             xfull, ctx_ref, partial, rs, ag_ssem, ag_rsem, rs_ssem, rs_rsem):
        my = lax.axis_index("i")
        left = (my - 1) % N_DEV
        right = (my + 1) % N_DEV

        xfull[:, pl.ds(my * SP, SP), :] = x_ref[...].astype(jnp.bfloat16)

        barrier_sem = pltpu.get_barrier_semaphore()
        for nbr in (left, right):
            pl.semaphore_signal(
                barrier_sem, inc=1,
                device_id=(nbr,), device_id_type=pl.DeviceIdType.MESH,
            )
        pl.semaphore_wait(barrier_sem, 2)

        for h in range(N_DEV - 1):
            o = (my - h) % N_DEV
            rdma = pltpu.make_async_remote_copy(
                src_ref=xfull.at[:, pl.ds(o * SP, SP), :],
                dst_ref=xfull.at[:, pl.ds(o * SP, SP), :],
                send_sem=ag_ssem.at[h],
                recv_sem=ag_rsem.at[h],
                device_id=(right,),
                device_id_type=pl.DeviceIdType.MESH,
            )
            rdma.start()
            rdma.wait()

        xf = xfull[...].reshape(B * S, D)
        wq = wq_ref[...].astype(jnp.bfloat16)
        wk = wk_ref[...].astype(jnp.bfloat16)
        wv = wv_ref[...].astype(jnp.bfloat16)
        q = jnp.dot(xf, wq, preferred_element_type=jnp.float32)
        k = jnp.dot(xf, wk, preferred_element_type=jnp.float32)
        v = jnp.dot(xf, wv, preferred_element_type=jnp.float32)
        q4 = _rope(q.reshape(B * S, H, DH)).astype(jnp.bfloat16)
        k4 = _rope(k.reshape(B * S, H, DH)).astype(jnp.bfloat16)
        v4 = v.reshape(B * S, H, DH).astype(jnp.bfloat16)

        for b in range(B):
            rows = slice(b * S, (b + 1) * S)
            for h in range(H):
                q_bh = q4[rows, h, :]
                k_bh = k4[rows, h, :]
                v_bh = v4[rows, h, :]
                s = lax.dot_general(
                    q_bh, k_bh, (((1,), (1,)), ((), ())),
                    preferred_element_type=jnp.float32,
                ) * 0.125
                m = jnp.max(s, axis=-1, keepdims=True)
                w = jnp.exp(s - m)
                w = w / jnp.sum(w, axis=-1, keepdims=True)
                ctx_bh = jnp.dot(
                    w.astype(jnp.bfloat16), v_bh,
                    preferred_element_type=jnp.float32,
                )
                ctx_ref[rows, h, :] = ctx_bh.astype(jnp.bfloat16)

        wo = wo_ref[...].astype(jnp.bfloat16)
        co = ctx_ref[...].reshape(B * S, H * DH)
        partial[...] = jnp.dot(
            co, wo, preferred_element_type=jnp.float32
        ).reshape(B, S, D)

        for st in range(N_DEV - 1):
            c_send = (my - 1 - st) % N_DEV
            if st == 0:
                rs[3] = partial[:, pl.ds(c_send * SP, SP), :]
            rdma = pltpu.make_async_remote_copy(
                src_ref=rs.at[3],
                dst_ref=rs.at[st],
                send_sem=rs_ssem.at[st],
                recv_sem=rs_rsem.at[st],
                device_id=(right,),
                device_id_type=pl.DeviceIdType.MESH,
            )
            rdma.start()
            rdma.wait()
            c_recv = (my - 2 - st) % N_DEV
            acc = rs[st] + partial[:, pl.ds(c_recv * SP, SP), :]
            if st < N_DEV - 2:
                rs[3] = acc
            else:
                out_ref[...] = acc

    return pl.pallas_call(
        body,
        out_shape=jax.ShapeDtypeStruct((B, SP, D), jnp.float32),
        in_specs=[pl.BlockSpec(memory_space=pltpu.VMEM)] * 5,
        out_specs=pl.BlockSpec(memory_space=pltpu.VMEM),
        scratch_shapes=[
            pltpu.VMEM((B, S, D), jnp.bfloat16),
            pltpu.VMEM((B * S, H, DH), jnp.bfloat16),
            pltpu.VMEM((B, S, D), jnp.float32),
            pltpu.VMEM((N_DEV, B, SP, D), jnp.float32),
            pltpu.SemaphoreType.DMA((N_DEV - 1,)),
            pltpu.SemaphoreType.DMA((N_DEV - 1,)),
            pltpu.SemaphoreType.DMA((N_DEV - 1,)),
            pltpu.SemaphoreType.DMA((N_DEV - 1,)),
        ],
        compiler_params=pltpu.CompilerParams(collective_id=0),
    )(x, Wq, Wk, Wv, Wo)


# baseline (device time: 132785 ns/iter reference)
import functools

import jax
import jax.numpy as jnp
from jax import lax
from jax.experimental import pallas as pl
from jax.experimental.pallas import tpu as pltpu

N_DEV = 4
B = 2
S = 1024
SP = S // N_DEV
D = 768
H = 4
DH = 64


def _rope(t):
    R, C = t.shape
    lane = lax.broadcasted_iota(jnp.int32, (R, C), 1)
    pos = (lax.broadcasted_iota(jnp.int32, (R, C), 0) % S).astype(jnp.float32)
    j = ((lane % DH) // 2).astype(jnp.float32)
    inv = jnp.exp(-jnp.log(10000.0) * (2.0 * j) / DH)
    ang = pos * inv
    cos = jnp.cos(ang)
    sin = jnp.sin(ang)
    tm1 = jnp.concatenate([t[:, 1:], t[:, :1]], axis=-1)
    tp1 = jnp.concatenate([t[:, -1:], t[:, :-1]], axis=-1)
    even = (lane % 2) == 0
    t_r = jnp.where(even, -tm1, tp1)
    return t * cos + t_r * sin


def kernel(x, Wq, Wk, Wv, Wo):
    def body(x_ref, wq_ref, wk_ref, wv_ref, wo_ref, out_ref,
             xfull, q_ref, k_ref, v_ref, ctx_ref, partial, rs,
             ag_ssem, ag_rsem, rs_ssem, rs_rsem):
        my = lax.axis_index("i")
        left = (my - 1) % N_DEV
        right = (my + 1) % N_DEV

        xfull[:, pl.ds(my * SP, SP), :] = x_ref[...].astype(jnp.bfloat16)

        barrier_sem = pltpu.get_barrier_semaphore()
        for nbr in (left, right):
            pl.semaphore_signal(
                barrier_sem, inc=1,
                device_id=(nbr,), device_id_type=pl.DeviceIdType.MESH,
            )
        pl.semaphore_wait(barrier_sem, 2)

        for h in range(N_DEV - 1):
            o = (my - h) % N_DEV
            rdma = pltpu.make_async_remote_copy(
                src_ref=xfull.at[:, pl.ds(o * SP, SP), :],
                dst_ref=xfull.at[:, pl.ds(o * SP, SP), :],
                send_sem=ag_ssem.at[h],
                recv_sem=ag_rsem.at[h],
                device_id=(right,),
                device_id_type=pl.DeviceIdType.MESH,
            )
            rdma.start()
            rdma.wait()

        xf = xfull[...].reshape(B * S, D)
        q_val = _rope(jnp.dot(xf, wq_ref[...].astype(jnp.bfloat16),
                              preferred_element_type=jnp.float32))
        for h in range(H):
            q_ref[h] = q_val[:, h * DH:(h + 1) * DH].astype(jnp.bfloat16)
        k_val = _rope(jnp.dot(xf, wk_ref[...].astype(jnp.bfloat16),
                              preferred_element_type=jnp.float32))
        for h in range(H):
            k_ref[h] = k_val[:, h * DH:(h + 1) * DH].astype(jnp.bfloat16)
        v_val = jnp.dot(xf, wv_ref[...].astype(jnp.bfloat16),
                        preferred_element_type=jnp.float32)
        for h in range(H):
            v_ref[h] = v_val[:, h * DH:(h + 1) * DH].astype(jnp.bfloat16)

        for b in range(B):
            rows = slice(b * S, (b + 1) * S)

            def attn_step(h, _, rows=rows):
                q_bh = q_ref[h, rows, :]
                k_bh = k_ref[h, rows, :]
                v_bh = v_ref[h, rows, :]
                s = lax.dot_general(
                    q_bh, k_bh, (((1,), (1,)), ((), ())),
                    preferred_element_type=jnp.float32,
                ) * 0.125
                m = jnp.max(s, axis=-1, keepdims=True)
                w = jnp.exp(s - m)
                w = w / jnp.sum(w, axis=-1, keepdims=True)
                ctx_bh = jnp.dot(
                    w.astype(jnp.bfloat16), v_bh,
                    preferred_element_type=jnp.float32,
                )
                ctx_ref[h, rows, :] = ctx_bh.astype(jnp.bfloat16)
                return _

            lax.fori_loop(0, H, attn_step, None)

        co = jnp.concatenate([ctx_ref[h] for h in range(H)], axis=-1)
        partial[...] = jnp.dot(
            co, wo_ref[...].astype(jnp.bfloat16),
            preferred_element_type=jnp.float32,
        ).reshape(B, S, D)

        for st in range(N_DEV - 1):
            c_send = (my - 1 - st) % N_DEV
            if st == 0:
                rs[3] = partial[:, pl.ds(c_send * SP, SP), :]
            rdma = pltpu.make_async_remote_copy(
                src_ref=rs.at[3],
                dst_ref=rs.at[st],
                send_sem=rs_ssem.at[st],
                recv_sem=rs_rsem.at[st],
                device_id=(right,),
                device_id_type=pl.DeviceIdType.MESH,
            )
            rdma.start()
            rdma.wait()
            c_recv = (my - 2 - st) % N_DEV
            acc = rs[st] + partial[:, pl.ds(c_recv * SP, SP), :]
            if st < N_DEV - 2:
                rs[3] = acc
            else:
                out_ref[...] = acc

    return pl.pallas_call(
        body,
        out_shape=jax.ShapeDtypeStruct((B, SP, D), jnp.float32),
        in_specs=[pl.BlockSpec(memory_space=pltpu.VMEM)] * 5,
        out_specs=pl.BlockSpec(memory_space=pltpu.VMEM),
        scratch_shapes=[
            pltpu.VMEM((B, S, D), jnp.bfloat16),
            pltpu.VMEM((H, B * S, DH), jnp.bfloat16),
            pltpu.VMEM((H, B * S, DH), jnp.bfloat16),
            pltpu.VMEM((H, B * S, DH), jnp.bfloat16),
            pltpu.VMEM((H, B * S, DH), jnp.bfloat16),
            pltpu.VMEM((B, S, D), jnp.float32),
            pltpu.VMEM((N_DEV, B, SP, D), jnp.float32),
            pltpu.SemaphoreType.DMA((N_DEV - 1,)),
            pltpu.SemaphoreType.DMA((N_DEV - 1,)),
            pltpu.SemaphoreType.DMA((N_DEV - 1,)),
            pltpu.SemaphoreType.DMA((N_DEV - 1,)),
        ],
        compiler_params=pltpu.CompilerParams(
            collective_id=0,
            vmem_limit_bytes=60 * 1024 * 1024,
        ),
    )(x, Wq, Wk, Wv, Wo)


# device time: 85558 ns/iter; 1.5520x vs baseline; 1.5520x over previous
import functools

import jax
import jax.numpy as jnp
from jax import lax
from jax.experimental import pallas as pl
from jax.experimental.pallas import tpu as pltpu

N_DEV = 4
B = 2
S = 1024
SP = S // N_DEV
D = 768
H = 4
DH = 64


def _rope(t):
    R, C = t.shape
    lane = lax.broadcasted_iota(jnp.int32, (R, C), 1)
    pos = (lax.broadcasted_iota(jnp.int32, (R, C), 0) % S).astype(jnp.float32)
    j = ((lane % DH) // 2).astype(jnp.float32)
    inv = jnp.exp(-jnp.log(10000.0) * (2.0 * j) / DH)
    ang = pos * inv
    cos = jnp.cos(ang)
    sin = jnp.sin(ang)
    tm1 = jnp.concatenate([t[:, 1:], t[:, :1]], axis=-1)
    tp1 = jnp.concatenate([t[:, -1:], t[:, :-1]], axis=-1)
    even = (lane % 2) == 0
    t_r = jnp.where(even, -tm1, tp1)
    return t * cos + t_r * sin


def kernel(x, Wq, Wk, Wv, Wo):
    def body(x_ref, wq_ref, wk_ref, wv_ref, wo_ref, out_ref,
             xfull, q_ref, k_ref, v_ref, ctx_ref, partial, rs_recv,
             ag_ssem, ag_rsem, rs_ssem, rs_rsem):
        my = lax.axis_index("i")
        left = (my - 1) % N_DEV
        right = (my + 1) % N_DEV
        opp = (my + 2) % N_DEV

        xfull[:, pl.ds(my * SP, SP), :] = x_ref[...].astype(jnp.bfloat16)

        barrier_sem = pltpu.get_barrier_semaphore()
        for nbr in (left, right, opp):
            pl.semaphore_signal(
                barrier_sem, inc=1,
                device_id=(nbr,), device_id_type=pl.DeviceIdType.MESH,
            )
        pl.semaphore_wait(barrier_sem, 3)

        ag_sends = []
        for idx, tgt in enumerate((right, left, opp)):
            r = pltpu.make_async_remote_copy(
                src_ref=xfull.at[:, pl.ds(my * SP, SP), :],
                dst_ref=xfull.at[:, pl.ds(my * SP, SP), :],
                send_sem=ag_ssem.at[idx],
                recv_sem=ag_rsem.at[idx],
                device_id=(tgt,),
                device_id_type=pl.DeviceIdType.MESH,
            )
            r.start()
            ag_sends.append(r)
        for idx, src_pos in enumerate((left, right, opp)):
            rd = pltpu.make_async_remote_copy(
                src_ref=xfull.at[:, pl.ds(my * SP, SP), :],
                dst_ref=xfull.at[:, pl.ds(src_pos * SP, SP), :],
                send_sem=ag_ssem.at[idx],
                recv_sem=ag_rsem.at[idx],
                device_id=(src_pos,),
                device_id_type=pl.DeviceIdType.MESH,
            )
            rd.wait_recv()
        for r in ag_sends:
            r.wait_send()

        xf = xfull[...].reshape(B * S, D)
        q_val = _rope(jnp.dot(xf, wq_ref[...].astype(jnp.bfloat16),
                              preferred_element_type=jnp.float32))
        for h in range(H):
            q_ref[h] = q_val[:, h * DH:(h + 1) * DH].astype(jnp.bfloat16)
        k_val = _rope(jnp.dot(xf, wk_ref[...].astype(jnp.bfloat16),
                              preferred_element_type=jnp.float32))
        for h in range(H):
            k_ref[h] = k_val[:, h * DH:(h + 1) * DH].astype(jnp.bfloat16)
        v_val = jnp.dot(xf, wv_ref[...].astype(jnp.bfloat16),
                        preferred_element_type=jnp.float32)
        for h in range(H):
            v_ref[h] = v_val[:, h * DH:(h + 1) * DH].astype(jnp.bfloat16)

        for b in range(B):
            rows = slice(b * S, (b + 1) * S)

            def attn_step(h, _, rows=rows):
                q_bh = q_ref[h, rows, :]
                k_bh = k_ref[h, rows, :]
                v_bh = v_ref[h, rows, :]
                s = lax.dot_general(
                    q_bh, k_bh, (((1,), (1,)), ((), ())),
                    preferred_element_type=jnp.float32,
                ) * 0.125
                m = jnp.max(s, axis=-1, keepdims=True)
                w = jnp.exp(s - m)
                w = w / jnp.sum(w, axis=-1, keepdims=True)
                ctx_bh = jnp.dot(
                    w.astype(jnp.bfloat16), v_bh,
                    preferred_element_type=jnp.float32,
                )
                ctx_ref[h, rows, :] = ctx_bh.astype(jnp.bfloat16)
                return _

            lax.fori_loop(0, H, attn_step, None)

        co = jnp.concatenate([ctx_ref[h] for h in range(H)], axis=-1)
        partial[...] = jnp.dot(
            co, wo_ref[...].astype(jnp.bfloat16),
            preferred_element_type=jnp.float32,
        ).reshape(B, S, D).astype(jnp.bfloat16)

        rs_sends = []
        for idx, tgt in enumerate((right, left, opp)):
            r = pltpu.make_async_remote_copy(
                src_ref=partial.at[:, pl.ds(tgt * SP, SP), :],
                dst_ref=rs_recv.at[idx],
                send_sem=rs_ssem.at[idx],
                recv_sem=rs_rsem.at[idx],
                device_id=(tgt,),
                device_id_type=pl.DeviceIdType.MESH,
            )
            r.start()
            rs_sends.append(r)
        for idx in range(3):
            rd = pltpu.make_async_remote_copy(
                src_ref=rs_recv.at[idx],
                dst_ref=rs_recv.at[idx],
                send_sem=rs_ssem.at[idx],
                recv_sem=rs_rsem.at[idx],
                device_id=(my,),
                device_id_type=pl.DeviceIdType.MESH,
            )
            rd.wait_recv()
        out_ref[...] = (
            partial[:, pl.ds(my * SP, SP), :].astype(jnp.float32)
            + rs_recv[0].astype(jnp.float32)
            + rs_recv[1].astype(jnp.float32)
            + rs_recv[2].astype(jnp.float32)
        )
        for r in rs_sends:
            r.wait_send()

    return pl.pallas_call(
        body,
        out_shape=jax.ShapeDtypeStruct((B, SP, D), jnp.float32),
        in_specs=[pl.BlockSpec(memory_space=pltpu.VMEM)] * 5,
        out_specs=pl.BlockSpec(memory_space=pltpu.VMEM),
        scratch_shapes=[
            pltpu.VMEM((B, S, D), jnp.bfloat16),
            pltpu.VMEM((H, B * S, DH), jnp.bfloat16),
            pltpu.VMEM((H, B * S, DH), jnp.bfloat16),
            pltpu.VMEM((H, B * S, DH), jnp.bfloat16),
            pltpu.VMEM((H, B * S, DH), jnp.bfloat16),
            pltpu.VMEM((B, S, D), jnp.bfloat16),
            pltpu.VMEM((3, B, SP, D), jnp.bfloat16),
            pltpu.SemaphoreType.DMA((N_DEV - 1,)),
            pltpu.SemaphoreType.DMA((N_DEV - 1,)),
            pltpu.SemaphoreType.DMA((N_DEV - 1,)),
            pltpu.SemaphoreType.DMA((N_DEV - 1,)),
        ],
        compiler_params=pltpu.CompilerParams(
            collective_id=0,
            vmem_limit_bytes=60 * 1024 * 1024,
        ),
    )(x, Wq, Wk, Wv, Wo)


# device time: 58168 ns/iter; 2.2828x vs baseline; 1.4709x over previous
import jax
import jax.numpy as jnp
from jax import lax
from jax.experimental import pallas as pl
from jax.experimental.pallas import tpu as pltpu

N_DEV = 4
B = 2
S = 1024
SP = S // N_DEV
D = 768
H = 4
DH = 64
DHA = 128
R = B * S


def kernel(x, Wq, Wk, Wv, Wo):
    def body(x_ref, wq_ref, wk_ref, wv_ref, wo_ref, out_ref,
             xfull, q_ref, k_ref, v_ref, ctxc_ref, partial, rs_recv,
             ag_ssem, ag_rsem, rs_ssem, rs_rsem):
        my = lax.axis_index("i")
        left = (my - 1) % N_DEV
        right = (my + 1) % N_DEV
        opp = (my + 2) % N_DEV

        xfull[:, pl.ds(my * SP, SP), :] = x_ref[...].astype(jnp.bfloat16)

        barrier_sem = pltpu.get_barrier_semaphore()
        for nbr in (left, right, opp):
            pl.semaphore_signal(
                barrier_sem, inc=1,
                device_id=(nbr,), device_id_type=pl.DeviceIdType.MESH,
            )
        pl.semaphore_wait(barrier_sem, 3)

        ag_sends = []
        for idx, tgt in enumerate((right, left, opp)):
            r = pltpu.make_async_remote_copy(
                src_ref=xfull.at[:, pl.ds(my * SP, SP), :],
                dst_ref=xfull.at[:, pl.ds(my * SP, SP), :],
                send_sem=ag_ssem.at[idx],
                recv_sem=ag_rsem.at[idx],
                device_id=(tgt,),
                device_id_type=pl.DeviceIdType.MESH,
            )
            r.start()
            ag_sends.append(r)

        lane = lax.broadcasted_iota(jnp.int32, (R, H * DH), 1)
        pos = (lax.broadcasted_iota(jnp.int32, (R, H * DH), 0) % S).astype(
            jnp.float32)
        j = ((lane % DH) // 2).astype(jnp.float32)
        inv = jnp.exp(-jnp.log(10000.0) * (2.0 * j) / DH)
        ang = pos * inv
        cos_t = jnp.cos(ang)
        sin_t = jnp.sin(ang)
        even = (lane % 2) == 0

        def rope(t):
            tm1 = jnp.concatenate([t[:, 1:], t[:, :1]], axis=-1)
            tp1 = jnp.concatenate([t[:, -1:], t[:, :-1]], axis=-1)
            t_r = jnp.where(even, -tm1, tp1)
            return t * cos_t + t_r * sin_t

        for idx, src_pos in enumerate((left, right, opp)):
            rd = pltpu.make_async_remote_copy(
                src_ref=xfull.at[:, pl.ds(my * SP, SP), :],
                dst_ref=xfull.at[:, pl.ds(src_pos * SP, SP), :],
                send_sem=ag_ssem.at[idx],
                recv_sem=ag_rsem.at[idx],
                device_id=(src_pos,),
                device_id_type=pl.DeviceIdType.MESH,
            )
            rd.wait_recv()
        for r in ag_sends:
            r.wait_send()

        xf = xfull[...].reshape(R, D)
        q_val = rope(jnp.dot(xf, wq_ref[...].astype(jnp.bfloat16),
                             preferred_element_type=jnp.float32)) * 0.125
        for h in range(H):
            q_ref[h] = q_val[:, h * DH:(h + 1) * DH].astype(jnp.bfloat16)
        k_val = rope(jnp.dot(xf, wk_ref[...].astype(jnp.bfloat16),
                             preferred_element_type=jnp.float32))
        for h in range(H):
            k_ref[h] = k_val[:, h * DH:(h + 1) * DH].astype(jnp.bfloat16)
        v_val = jnp.dot(xf, wv_ref[...].astype(jnp.bfloat16),
                        preferred_element_type=jnp.float32)
        ones_col = jnp.full((R, 1), 1.0, jnp.bfloat16)
        zeros_pad = jnp.zeros((R, DHA - DH - 1), jnp.bfloat16)
        for h in range(H):
            v_ref[h] = jnp.concatenate(
                [v_val[:, h * DH:(h + 1) * DH].astype(jnp.bfloat16),
                 ones_col, zeros_pad], axis=-1)

        wo_bf = wo_ref[...].astype(jnp.bfloat16)
        rs_sends = []

        def chunk_partial(tgt):
            for b in range(B):
                rows_full = slice(b * S, (b + 1) * S)

                def attn_h(h, _, rows_full=rows_full, b=b):
                    qc = q_ref[h, pl.ds(b * S + tgt * SP, SP), :]
                    kb = k_ref[h, rows_full, :]
                    s = lax.dot_general(
                        qc, kb, (((1,), (1,)), ((), ())),
                        preferred_element_type=jnp.float32,
                    )
                    w = jnp.exp(s.astype(jnp.bfloat16))
                    ca = jnp.dot(w, v_ref[h, rows_full, :],
                                 preferred_element_type=jnp.float32)
                    ctxc_ref[h, pl.ds(b * SP, SP), :] = ca
                    return _

                lax.fori_loop(0, H, attn_h, None)
            cols = []
            for h in range(H):
                ca = ctxc_ref[h]
                cols.append(
                    (ca[:, :DH] / ca[:, DH:DH + 1]).astype(jnp.bfloat16))
            co = jnp.concatenate(cols, axis=-1)
            return jnp.dot(co, wo_bf, preferred_element_type=jnp.float32)

        for idx, tgt in ((2, opp), (0, right), (1, left)):
            pc = chunk_partial(tgt).astype(jnp.bfloat16).reshape(B, SP, D)
            partial[:, pl.ds(tgt * SP, SP), :] = pc
            r = pltpu.make_async_remote_copy(
                src_ref=partial.at[:, pl.ds(tgt * SP, SP), :],
                dst_ref=rs_recv.at[idx],
                send_sem=rs_ssem.at[idx],
                recv_sem=rs_rsem.at[idx],
                device_id=(tgt,),
                device_id_type=pl.DeviceIdType.MESH,
            )
            r.start()
            rs_sends.append(r)

        mine = chunk_partial(my).reshape(B, SP, D)

        for idx in range(3):
            rd = pltpu.make_async_remote_copy(
                src_ref=rs_recv.at[idx],
                dst_ref=rs_recv.at[idx],
                send_sem=rs_ssem.at[idx],
                recv_sem=rs_rsem.at[idx],
                device_id=(my,),
                device_id_type=pl.DeviceIdType.MESH,
            )
            rd.wait_recv()
        out_ref[...] = (
            mine
            + rs_recv[0].astype(jnp.float32)
            + rs_recv[1].astype(jnp.float32)
            + rs_recv[2].astype(jnp.float32)
        )
        for r in rs_sends:
            r.wait_send()

    return pl.pallas_call(
        body,
        out_shape=jax.ShapeDtypeStruct((B, SP, D), jnp.float32),
        in_specs=[pl.BlockSpec(memory_space=pltpu.VMEM)] * 5,
        out_specs=pl.BlockSpec(memory_space=pltpu.VMEM),
        scratch_shapes=[
            pltpu.VMEM((B, S, D), jnp.bfloat16),
            pltpu.VMEM((H, R, DH), jnp.bfloat16),
            pltpu.VMEM((H, R, DH), jnp.bfloat16),
            pltpu.VMEM((H, R, DHA), jnp.bfloat16),
            pltpu.VMEM((H, B * SP, DHA), jnp.float32),
            pltpu.VMEM((B, S, D), jnp.bfloat16),
            pltpu.VMEM((3, B, SP, D), jnp.bfloat16),
            pltpu.SemaphoreType.DMA((3,)),
            pltpu.SemaphoreType.DMA((3,)),
            pltpu.SemaphoreType.DMA((3,)),
            pltpu.SemaphoreType.DMA((3,)),
        ],
        compiler_params=pltpu.CompilerParams(
            collective_id=0,
            vmem_limit_bytes=60 * 1024 * 1024,
        ),
    )(x, Wq, Wk, Wv, Wo)


# device time: 56467 ns/iter; 2.3516x vs baseline; 1.0301x over previous
import jax
import jax.numpy as jnp
from jax import lax
from jax.experimental import pallas as pl
from jax.experimental.pallas import tpu as pltpu

N_DEV = 4
B = 2
S = 1024
SP = S // N_DEV
D = 768
H = 4
DH = 64
DHA = 128
R = B * S


def kernel(x, Wq, Wk, Wv, Wo):
    def body(x_ref, wq_ref, wk_ref, wv_ref, wo_ref, out_ref,
             xfull, q_ref, k_ref, v_ref, ctxc_ref, partial, rs_recv,
             ag_ssem, ag_rsem, rs_ssem, rs_rsem):
        my = lax.axis_index("i")
        left = (my - 1) % N_DEV
        right = (my + 1) % N_DEV
        opp = (my + 2) % N_DEV

        xfull[:, pl.ds(my * SP, SP), :] = x_ref[...].astype(jnp.bfloat16)

        barrier_sem = pltpu.get_barrier_semaphore()
        for nbr in (left, right, opp):
            pl.semaphore_signal(
                barrier_sem, inc=1,
                device_id=(nbr,), device_id_type=pl.DeviceIdType.MESH,
            )
        pl.semaphore_wait(barrier_sem, 3)

        ag_sends = []
        for idx, tgt in enumerate((right, left, opp)):
            r = pltpu.make_async_remote_copy(
                src_ref=xfull.at[:, pl.ds(my * SP, SP), :],
                dst_ref=xfull.at[:, pl.ds(my * SP, SP), :],
                send_sem=ag_ssem.at[idx],
                recv_sem=ag_rsem.at[idx],
                device_id=(tgt,),
                device_id_type=pl.DeviceIdType.MESH,
            )
            r.start()
            ag_sends.append(r)

        RC = B * SP
        lane = lax.broadcasted_iota(jnp.int32, (RC, H * DH), 1)
        srow = (lax.broadcasted_iota(jnp.int32, (RC, H * DH), 0) % SP).astype(
            jnp.float32)
        j = ((lane % DH) // 2).astype(jnp.float32)
        inv = jnp.exp(-jnp.log(10000.0) * (2.0 * j) / DH)
        even = (lane % 2) == 0
        wq_bf = wq_ref[...].astype(jnp.bfloat16)
        wk_bf = wk_ref[...].astype(jnp.bfloat16)
        wv_bf = wv_ref[...].astype(jnp.bfloat16)
        ones_col = jnp.full((RC, 1), 1.0, jnp.bfloat16)
        zeros_pad = jnp.zeros((RC, DHA - DH - 1), jnp.bfloat16)

        def qkv_chunk(c):
            ang = (srow + c.astype(jnp.float32) * SP) * inv
            cos_t = jnp.cos(ang)
            sin_t = jnp.sin(ang)

            def rope(t):
                tm1 = jnp.concatenate([t[:, 1:], t[:, :1]], axis=-1)
                tp1 = jnp.concatenate([t[:, -1:], t[:, :-1]], axis=-1)
                t_r = jnp.where(even, -tm1, tp1)
                return t * cos_t + t_r * sin_t

            xc = xfull[:, pl.ds(c * SP, SP), :].reshape(RC, D)
            q_c = (rope(jnp.dot(xc, wq_bf,
                                preferred_element_type=jnp.float32))
                   * 0.125).astype(jnp.bfloat16)
            k_c = rope(jnp.dot(xc, wk_bf,
                               preferred_element_type=jnp.float32)
                       ).astype(jnp.bfloat16)
            v_c = jnp.dot(xc, wv_bf,
                          preferred_element_type=jnp.float32)
            for h in range(H):
                cols = slice(h * DH, (h + 1) * DH)
                va = jnp.concatenate(
                    [v_c[:, cols].astype(jnp.bfloat16), ones_col, zeros_pad],
                    axis=-1)
                for b in range(B):
                    brows = slice(b * SP, (b + 1) * SP)
                    rows = pl.ds(b * S + c * SP, SP)
                    q_ref[h, rows, :] = q_c[brows, cols]
                    k_ref[h, rows, :] = k_c[brows, cols]
                    v_ref[h, rows, :] = va[brows, :]

        qkv_chunk(my)
        for idx, src_pos in enumerate((left, right, opp)):
            rd = pltpu.make_async_remote_copy(
                src_ref=xfull.at[:, pl.ds(my * SP, SP), :],
                dst_ref=xfull.at[:, pl.ds(src_pos * SP, SP), :],
                send_sem=ag_ssem.at[idx],
                recv_sem=ag_rsem.at[idx],
                device_id=(src_pos,),
                device_id_type=pl.DeviceIdType.MESH,
            )
            rd.wait_recv()
            qkv_chunk(src_pos)
        for r in ag_sends:
            r.wait_send()

        wo_bf = wo_ref[...].astype(jnp.bfloat16)
        rs_sends = []

        def chunk_partial(tgt):
            for b in range(B):
                rows_full = slice(b * S, (b + 1) * S)

                def attn_h(h, _, rows_full=rows_full, b=b):
                    qc = q_ref[h, pl.ds(b * S + tgt * SP, SP), :]
                    kb = k_ref[h, rows_full, :]
                    s = lax.dot_general(
                        qc, kb, (((1,), (1,)), ((), ())),
                        preferred_element_type=jnp.float32,
                    )
                    w = jnp.exp(s.astype(jnp.bfloat16))
                    ca = jnp.dot(w, v_ref[h, rows_full, :],
                                 preferred_element_type=jnp.float32)
                    ctxc_ref[h, pl.ds(b * SP, SP), :] = ca
                    return _

                lax.fori_loop(0, H, attn_h, None)
            cols = []
            for h in range(H):
                ca = ctxc_ref[h]
                cols.append(
                    (ca[:, :DH] / ca[:, DH:DH + 1]).astype(jnp.bfloat16))
            co = jnp.concatenate(cols, axis=-1)
            return jnp.dot(co, wo_bf, preferred_element_type=jnp.float32)

        for idx, tgt in ((2, opp), (0, right), (1, left)):
            pc = chunk_partial(tgt).astype(jnp.bfloat16).reshape(B, SP, D)
            partial[:, pl.ds(tgt * SP, SP), :] = pc
            r = pltpu.make_async_remote_copy(
                src_ref=partial.at[:, pl.ds(tgt * SP, SP), :],
                dst_ref=rs_recv.at[idx],
                send_sem=rs_ssem.at[idx],
                recv_sem=rs_rsem.at[idx],
                device_id=(tgt,),
                device_id_type=pl.DeviceIdType.MESH,
            )
            r.start()
            rs_sends.append(r)

        mine = chunk_partial(my).reshape(B, SP, D)

        for idx in range(3):
            rd = pltpu.make_async_remote_copy(
                src_ref=rs_recv.at[idx],
                dst_ref=rs_recv.at[idx],
                send_sem=rs_ssem.at[idx],
                recv_sem=rs_rsem.at[idx],
                device_id=(my,),
                device_id_type=pl.DeviceIdType.MESH,
            )
            rd.wait_recv()
        out_ref[...] = (
            mine
            + rs_recv[0].astype(jnp.float32)
            + rs_recv[1].astype(jnp.float32)
            + rs_recv[2].astype(jnp.float32)
        )
        for r in rs_sends:
            r.wait_send()

    return pl.pallas_call(
        body,
        out_shape=jax.ShapeDtypeStruct((B, SP, D), jnp.float32),
        in_specs=[pl.BlockSpec(memory_space=pltpu.VMEM)] * 5,
        out_specs=pl.BlockSpec(memory_space=pltpu.VMEM),
        scratch_shapes=[
            pltpu.VMEM((B, S, D), jnp.bfloat16),
            pltpu.VMEM((H, R, DH), jnp.bfloat16),
            pltpu.VMEM((H, R, DH), jnp.bfloat16),
            pltpu.VMEM((H, R, DHA), jnp.bfloat16),
            pltpu.VMEM((H, B * SP, DHA), jnp.float32),
            pltpu.VMEM((B, S, D), jnp.bfloat16),
            pltpu.VMEM((3, B, SP, D), jnp.bfloat16),
            pltpu.SemaphoreType.DMA((3,)),
            pltpu.SemaphoreType.DMA((3,)),
            pltpu.SemaphoreType.DMA((3,)),
            pltpu.SemaphoreType.DMA((3,)),
        ],
        compiler_params=pltpu.CompilerParams(
            collective_id=0,
            vmem_limit_bytes=60 * 1024 * 1024,
        ),
    )(x, Wq, Wk, Wv, Wo)
